# initial kernel scaffold (unmeasured)
import jax
import jax.numpy as jnp
from jax import lax
from jax.experimental import pallas as pl
from jax.experimental.pallas import tpu as pltpu


def kernel(
    x,
):
    def body(*refs):
        pass

    out_shape = jax.ShapeDtypeStruct(..., jnp.float32)
    return pl.pallas_call(body, out_shape=out_shape)(...)


# baseline (device time: 215151 ns/iter reference)
import jax
import jax.numpy as jnp
from jax import lax
from jax.experimental import pallas as pl
from jax.experimental.pallas import tpu as pltpu


def kernel(x):
    m_per, n = x.shape
    half = m_per // 2

    def body(x_ref, out_ref, local_sem, send_sems, recv_sems):
        mx = lax.axis_index("x")
        my = lax.axis_index("y")

        barrier = pltpu.get_barrier_semaphore()
        pl.semaphore_signal(
            barrier, inc=1, device_id=(mx, 1 - my),
            device_id_type=pl.DeviceIdType.MESH,
        )
        pl.semaphore_signal(
            barrier, inc=1, device_id=(1 - mx, my),
            device_id_type=pl.DeviceIdType.MESH,
        )
        pl.semaphore_wait(barrier, 2)

        local = pltpu.make_async_copy(
            x_ref, out_ref.at[pl.ds(my * m_per, m_per), :], local_sem
        )
        local.start()

        p1 = pltpu.make_async_remote_copy(
            src_ref=x_ref.at[pl.ds(mx * half, half), :],
            dst_ref=out_ref.at[pl.ds(my * m_per + mx * half, half), :],
            send_sem=send_sems.at[0],
            recv_sem=recv_sems.at[0],
            device_id=(mx, 1 - my),
            device_id_type=pl.DeviceIdType.MESH,
        )
        p1.start()
        p1.wait()

        recv_row = (1 - my) * m_per + mx * half
        p2 = pltpu.make_async_remote_copy(
            src_ref=out_ref.at[pl.ds(recv_row, half), :],
            dst_ref=out_ref.at[pl.ds(recv_row, half), :],
            send_sem=send_sems.at[1],
            recv_sem=recv_sems.at[1],
            device_id=(1 - mx, my),
            device_id_type=pl.DeviceIdType.MESH,
        )
        p2.start()
        p2.wait()
        local.wait()

    return pl.pallas_call(
        body,
        out_shape=jax.ShapeDtypeStruct((2 * m_per, n), x.dtype),
        in_specs=[pl.BlockSpec(memory_space=pl.ANY)],
        out_specs=pl.BlockSpec(memory_space=pl.ANY),
        scratch_shapes=[
            pltpu.SemaphoreType.DMA,
            pltpu.SemaphoreType.DMA((2,)),
            pltpu.SemaphoreType.DMA((2,)),
        ],
        compiler_params=pltpu.CompilerParams(collective_id=0),
    )(x)


# device time: 131628 ns/iter; 1.6345x vs baseline; 1.6345x over previous
import jax
import jax.numpy as jnp
from jax import lax
from jax.experimental import pallas as pl
from jax.experimental.pallas import tpu as pltpu


N_CHUNKS = 16


def kernel(x):
    m_per, n = x.shape
    half = m_per // 2
    chunk = half // N_CHUNKS

    def body(x_ref, out_ref, local_sem, s1, r1, s2, r2):
        mx = lax.axis_index("x")
        my = lax.axis_index("y")

        barrier = pltpu.get_barrier_semaphore()
        pl.semaphore_signal(
            barrier, inc=1, device_id=(mx, 1 - my),
            device_id_type=pl.DeviceIdType.MESH,
        )
        pl.semaphore_signal(
            barrier, inc=1, device_id=(1 - mx, my),
            device_id_type=pl.DeviceIdType.MESH,
        )
        pl.semaphore_wait(barrier, 2)

        local = pltpu.make_async_copy(
            x_ref, out_ref.at[pl.ds(my * m_per, m_per), :], local_sem
        )
        local.start()

        p1 = []
        for k in range(N_CHUNKS):
            row = mx * half + k * chunk
            d = pltpu.make_async_remote_copy(
                src_ref=x_ref.at[pl.ds(row, chunk), :],
                dst_ref=out_ref.at[pl.ds(my * m_per + row, chunk), :],
                send_sem=s1.at[k],
                recv_sem=r1.at[k],
                device_id=(mx, 1 - my),
                device_id_type=pl.DeviceIdType.MESH,
            )
            d.start()
            p1.append(d)

        p2 = []
        for k in range(N_CHUNKS):
            p1[k].wait_recv()
            row = (1 - my) * m_per + mx * half + k * chunk
            d = pltpu.make_async_remote_copy(
                src_ref=out_ref.at[pl.ds(row, chunk), :],
                dst_ref=out_ref.at[pl.ds(row, chunk), :],
                send_sem=s2.at[k],
                recv_sem=r2.at[k],
                device_id=(1 - mx, my),
                device_id_type=pl.DeviceIdType.MESH,
            )
            d.start()
            p2.append(d)

        for k in range(N_CHUNKS):
            p2[k].wait_recv()
            p1[k].wait_send()
            p2[k].wait_send()
        local.wait()

    return pl.pallas_call(
        body,
        out_shape=jax.ShapeDtypeStruct((2 * m_per, n), x.dtype),
        in_specs=[pl.BlockSpec(memory_space=pl.ANY)],
        out_specs=pl.BlockSpec(memory_space=pl.ANY),
        scratch_shapes=[
            pltpu.SemaphoreType.DMA,
            pltpu.SemaphoreType.DMA((N_CHUNKS,)),
            pltpu.SemaphoreType.DMA((N_CHUNKS,)),
            pltpu.SemaphoreType.DMA((N_CHUNKS,)),
            pltpu.SemaphoreType.DMA((N_CHUNKS,)),
        ],
        compiler_params=pltpu.CompilerParams(collective_id=0),
    )(x)


# device time: 129976 ns/iter; 1.6553x vs baseline; 1.0127x over previous
import jax
import jax.numpy as jnp
from jax import lax
from jax.experimental import pallas as pl
from jax.experimental.pallas import tpu as pltpu

N_CHUNKS = 32
CHUNK_ROWS = (2048 // N_CHUNKS,) * N_CHUNKS
CHUNK_OFF = tuple(sum(CHUNK_ROWS[:k]) for k in range(N_CHUNKS))


def kernel(x):
    m_per, n = x.shape
    half = m_per // 2
    assert sum(CHUNK_ROWS) == half

    def body(x_ref, out_ref, local_sem, s1, r1, s2, r2):
        mx = lax.axis_index("x")
        my = lax.axis_index("y")

        barrier = pltpu.get_barrier_semaphore()
        pl.semaphore_signal(
            barrier, inc=1, device_id=(mx, 1 - my),
            device_id_type=pl.DeviceIdType.MESH,
        )
        pl.semaphore_signal(
            barrier, inc=1, device_id=(1 - mx, my),
            device_id_type=pl.DeviceIdType.MESH,
        )
        pl.semaphore_wait(barrier, 2)

        local = pltpu.make_async_copy(
            x_ref, out_ref.at[pl.ds(my * m_per, m_per), :], local_sem
        )
        local.start()

        p1 = []
        for k in range(N_CHUNKS):
            row = mx * half + CHUNK_OFF[k]
            d = pltpu.make_async_remote_copy(
                src_ref=x_ref.at[pl.ds(row, CHUNK_ROWS[k]), :],
                dst_ref=out_ref.at[pl.ds(my * m_per + row, CHUNK_ROWS[k]), :],
                send_sem=s1.at[k],
                recv_sem=r1.at[k],
                device_id=(mx, 1 - my),
                device_id_type=pl.DeviceIdType.MESH,
            )
            d.start()
            p1.append(d)

        p2 = []
        for k in range(N_CHUNKS):
            p1[k].wait_recv()
            row = (1 - my) * m_per + mx * half + CHUNK_OFF[k]
            d = pltpu.make_async_remote_copy(
                src_ref=out_ref.at[pl.ds(row, CHUNK_ROWS[k]), :],
                dst_ref=out_ref.at[pl.ds(row, CHUNK_ROWS[k]), :],
                send_sem=s2.at[k],
                recv_sem=r2.at[k],
                device_id=(1 - mx, my),
                device_id_type=pl.DeviceIdType.MESH,
            )
            d.start()
            p2.append(d)

        for k in range(N_CHUNKS):
            p2[k].wait_recv()
            p1[k].wait_send()
            p2[k].wait_send()
        local.wait()

    return pl.pallas_call(
        body,
        out_shape=jax.ShapeDtypeStruct((2 * m_per, n), x.dtype),
        in_specs=[pl.BlockSpec(memory_space=pl.ANY)],
        out_specs=pl.BlockSpec(memory_space=pl.ANY),
        scratch_shapes=[
            pltpu.SemaphoreType.DMA,
            pltpu.SemaphoreType.DMA((N_CHUNKS,)),
            pltpu.SemaphoreType.DMA((N_CHUNKS,)),
            pltpu.SemaphoreType.DMA((N_CHUNKS,)),
            pltpu.SemaphoreType.DMA((N_CHUNKS,)),
        ],
        compiler_params=pltpu.CompilerParams(collective_id=0),
    )(x)
